# baseline (speedup 1.0000x reference)
import functools

import jax
import jax.numpy as jnp
from jax import lax
from jax.experimental import pallas as pl
from jax.experimental.pallas import tpu as pltpu
import jax.experimental.pallas.tpu_sc as plsc

NC = 2
NS = 16
L = 16
NW = NC * NS

CHUNK = 128

_MESH = dict(core_axis_name="c", subcore_axis_name="s", num_cores=NC,
             num_subcores=NS)


def _tc_front(feature, W1, W2, Wlin, src, dst, adv, n_pad, e_pad):
    n = feature.shape[0]
    d1 = W1.shape[1]
    d1h = d1 // 2
    e = src.shape[0]

    def body(f_ref, w_ref, w2_ref, wl_ref, s_ref, d_ref, a_ref,
             sup_ref, w2l_ref, so_ref, do_ref, ao_ref):
        sup = jnp.dot(f_ref[...], w_ref[...],
                      preferred_element_type=jnp.float32)
        z = jnp.zeros((n_pad - n, d1h), jnp.float32)
        sup_ref[pl.ds(0, n), :] = sup[:, :d1h]
        sup_ref[pl.ds(n, n_pad - n), :] = z
        sup_ref[pl.ds(n_pad, n), :] = sup[:, d1h:]
        sup_ref[pl.ds(n_pad + n, n_pad - n), :] = z
        w2l_ref[...] = jnp.dot(w2_ref[...], wl_ref[...],
                               preferred_element_type=jnp.float32)[:, 0]
        so_ref[pl.ds(0, e)] = s_ref[...]
        so_ref[pl.ds(e, e_pad - e)] = jnp.zeros((e_pad - e,), jnp.int32)
        do_ref[pl.ds(0, e)] = d_ref[...]
        do_ref[pl.ds(e, e_pad - e)] = jnp.zeros((e_pad - e,), jnp.int32)
        ao_ref[pl.ds(0, e)] = a_ref[...]
        ao_ref[pl.ds(e, e_pad - e)] = jnp.zeros((e_pad - e,), jnp.float32)

    return pl.pallas_call(
        body,
        out_shape=(
            jax.ShapeDtypeStruct((NC * n_pad, d1h), jnp.float32),
            jax.ShapeDtypeStruct((d1,), jnp.float32),
            jax.ShapeDtypeStruct((e_pad,), jnp.int32),
            jax.ShapeDtypeStruct((e_pad,), jnp.int32),
            jax.ShapeDtypeStruct((e_pad,), jnp.float32),
        ),
    )(feature, W1, W2, Wlin, src, dst, adv)


def _tc_final(q, b2, Wlin, blin, n):
    def body(q_ref, b2_ref, wl_ref, bl_ref, o_ref):
        c = jnp.dot(b2_ref[...][None, :], wl_ref[...],
                    preferred_element_type=jnp.float32)[0, 0] + bl_ref[0]
        s = q_ref[0, :n] + q_ref[1, :n] + c
        o_ref[...] = jax.nn.sigmoid(s)[:, None]
    return pl.pallas_call(
        body,
        out_shape=jax.ShapeDtypeStruct((n, 1), jnp.float32),
    )(q, b2, Wlin, blin)


def _sc_spmm_wide(src3, dst3, adj3, sup2, b1, w2l, n_pad, d1h, nchunk):
    rows_per_tile = n_pad // NS
    assert rows_per_tile % CHUNK == 0
    assert nchunk % 4 == 0

    @functools.partial(
        pl.kernel,
        out_type=jax.ShapeDtypeStruct((NC * n_pad,), jnp.float32),
        mesh=plsc.VectorSubcoreMesh(**_MESH),
        compiler_params=pltpu.CompilerParams(use_tc_tiling_on_sc=False,
                                             needs_layout_passes=False),
        scratch_types=[
            pltpu.VMEM((nchunk, CHUNK), jnp.int32),
            pltpu.VMEM((nchunk, CHUNK), jnp.int32),
            pltpu.VMEM((nchunk, CHUNK), jnp.float32),
            pltpu.VMEM((CHUNK, d1h), jnp.float32),
            pltpu.VMEM((CHUNK, d1h), jnp.float32),
            pltpu.VMEM((CHUNK, d1h), jnp.float32),
            pltpu.VMEM((CHUNK, d1h), jnp.float32),
            pltpu.VMEM((NC * d1h,), jnp.float32),
            pltpu.VMEM((NC * d1h,), jnp.float32),
            pltpu.VMEM((d1h, L), jnp.float32),
            pltpu.VMEM((d1h, L), jnp.float32),
            pltpu.VMEM((rows_per_tile,), jnp.float32),
            pltpu.VMEM_SHARED((n_pad, d1h), jnp.float32),
            pltpu.VMEM_SHARED((n_pad, d1h), jnp.float32),
            pltpu.SemaphoreType.DMA,
            pltpu.SemaphoreType.DMA,
            pltpu.SemaphoreType.DMA,
            pltpu.SemaphoreType.DMA,
            pltpu.SemaphoreType.DMA,
            pltpu.SemaphoreType.DMA,
            pltpu.SemaphoreType.DMA,
            pltpu.SemaphoreType.DMA,
        ],
    )
    def spmm1(src_hbm, dst_hbm, adj_hbm, sup_hbm, b1_hbm, w2l_hbm, out_hbm,
              src_v, dst_v, adj_v, buf0, buf1, buf2, buf3,
              b1_v, w2l_v, bvec, wvec, vbuf, acc, sup_sh,
              gsem0, gsem1, gsem2, gsem3, ssem0, ssem1, ssem2, ssem3):
        c = lax.axis_index("c")
        s = lax.axis_index("s")
        d1 = d1h

        pltpu.sync_copy(
            sup_hbm.at[pl.ds(c * n_pad + s * rows_per_tile,
                             rows_per_tile)],
            sup_sh.at[pl.ds(s * rows_per_tile, rows_per_tile)])

        @pl.loop(0, CHUNK)
        def _zrow(r):
            for j in range(d1 // L):
                buf0.at[r][pl.ds(j * L, L)] = jnp.zeros((L,), jnp.float32)

        @pl.loop(0, rows_per_tile // CHUNK)
        def _zacc(i):
            pltpu.sync_copy(
                buf0, acc.at[pl.ds(s * rows_per_tile + i * CHUNK, CHUNK)])
        plsc.subcore_barrier()

        pltpu.sync_copy(src_hbm.at[s], src_v)
        pltpu.sync_copy(dst_hbm.at[s], dst_v)
        pltpu.sync_copy(adj_hbm.at[s], adj_v)

        bufs = (buf0, buf1, buf2, buf3)
        gsems = (gsem0, gsem1, gsem2, gsem3)
        ssems = (ssem0, ssem1, ssem2, ssem3)
        NBUF = 4

        bcast_dn = lax.GatherDimensionNumbers(
            offset_dims=(), collapsed_slice_dims=(0,), start_index_map=(0,))

        def scale(buf, ch):
            @pl.loop(0, CHUNK // L)
            def _scale(k):
                a16 = adj_v.at[ch][pl.ds(k * L, L)]
                for r2 in range(L):
                    av = lax.gather(
                        a16, jnp.full((L, 1), r2, jnp.int32), bcast_dn,
                        slice_sizes=(1,),
                        mode=lax.GatherScatterMode.PROMISE_IN_BOUNDS)
                    row = k * L + r2
                    for j in range(d1 // L):
                        buf.at[row][pl.ds(j * L, L)] = (
                            buf.at[row][pl.ds(j * L, L)] * av)

        def gather(ch, b):
            pltpu.async_copy(sup_sh.at[src_v.at[ch]], bufs[b], gsems[b])

        def gather_wait(ch, b):
            pltpu.make_async_copy(
                sup_sh.at[src_v.at[ch]], bufs[b], gsems[b]).wait()

        def scatter(ch, b):
            pltpu.async_copy(bufs[b], acc.at[dst_v.at[ch]], ssems[b],
                             add=True)

        def scatter_wait(ch, b):
            pltpu.make_async_copy(
                bufs[b], acc.at[dst_v.at[ch]], ssems[b]).wait()

        for b in range(NBUF - 1):
            gather(b, b)

        @pl.loop(0, nchunk // NBUF)
        def _edges(it):
            ch0 = it * NBUF
            for b in range(NBUF):
                ch = ch0 + b
                gather_wait(ch, b)
                scale(bufs[b], ch)
                scatter(ch, b)
                bn = (b + NBUF - 1) % NBUF

                @pl.when(ch > 0)
                def _():
                    pltpu.make_async_copy(
                        bufs[bn], acc.at[dst_v.at[ch - 1]],
                        ssems[bn]).wait()

                @pl.when(ch + NBUF - 1 < nchunk)
                def _():
                    pltpu.async_copy(
                        sup_sh.at[src_v.at[ch + NBUF - 1]], bufs[bn],
                        gsems[bn])

        scatter_wait(nchunk - 1, (nchunk - 1) % NBUF)

        pltpu.sync_copy(b1_hbm, b1_v)
        pltpu.sync_copy(w2l_hbm, w2l_v)
        for jj in range(d1h // L):
            b16 = b1_v[pl.ds(c * d1h + jj * L, L)]
            w16 = w2l_v[pl.ds(c * d1h + jj * L, L)]
            for t in range(L):
                idx = jnp.full((L, 1), t, jnp.int32)
                bvec.at[jj * L + t][pl.ds(0, L)] = lax.gather(
                    b16, idx, bcast_dn, slice_sizes=(1,),
                    mode=lax.GatherScatterMode.PROMISE_IN_BOUNDS)
                wvec.at[jj * L + t][pl.ds(0, L)] = lax.gather(
                    w16, idx, bcast_dn, slice_sizes=(1,),
                    mode=lax.GatherScatterMode.PROMISE_IN_BOUNDS)
        plsc.subcore_barrier()

        iota = lax.iota(jnp.int32, L)

        @pl.loop(0, rows_per_tile // CHUNK)
        def _vout(i):
            base = s * rows_per_tile + i * CHUNK
            pltpu.sync_copy(acc.at[pl.ds(base, CHUNK)], buf0)

            @pl.loop(0, CHUNK // L)
            def _vrows(g):
                rows = iota + g * L
                vacc = jnp.zeros((L,), jnp.float32)
                for j in range(d1h):
                    col = plsc.load_gather(
                        buf0, [rows, jnp.full((L,), j, jnp.int32)])
                    vacc = vacc + (
                        jnp.maximum(col + bvec.at[j][pl.ds(0, L)], 0.0)
                        * wvec.at[j][pl.ds(0, L)])
                vbuf[pl.ds(i * CHUNK + g * L, L)] = vacc

        pltpu.sync_copy(
            vbuf, out_hbm.at[pl.ds(c * n_pad + s * rows_per_tile,
                                   rows_per_tile)])

    return spmm1(src3, dst3, adj3, sup2, b1, w2l)


def _sc_spmm_scalar(src3, dst3, adj3, vh, n_pad, nchunk):
    rows_per_tile = n_pad // NS
    nch2 = nchunk // 2

    @functools.partial(
        pl.kernel,
        out_type=jax.ShapeDtypeStruct((NC, n_pad), jnp.float32),
        mesh=plsc.VectorSubcoreMesh(**_MESH),
        compiler_params=pltpu.CompilerParams(use_tc_tiling_on_sc=False,
                                             needs_layout_passes=False),
        scratch_types=[
            pltpu.VMEM((nch2, CHUNK), jnp.int32),
            pltpu.VMEM((nch2, CHUNK), jnp.int32),
            pltpu.VMEM((nch2, CHUNK), jnp.float32),
            pltpu.VMEM((nch2, CHUNK), jnp.float32),
            pltpu.VMEM((n_pad,), jnp.float32),
            pltpu.VMEM((n_pad,), jnp.float32),
            pltpu.VMEM((rows_per_tile,), jnp.float32),
            pltpu.VMEM_SHARED((n_pad,), jnp.float32),
        ],
    )
    def spmm2(src_hbm, dst_hbm, adj_hbm, vh_hbm, out_hbm,
              src_v, dst_v, adj_v, msg_v, vloc, vtmp, obuf, acc):
        c = lax.axis_index("c")
        s = lax.axis_index("s")

        @pl.loop(0, rows_per_tile // L)
        def _z(i):
            obuf[pl.ds(i * L, L)] = jnp.zeros((L,), jnp.float32)
        pltpu.sync_copy(obuf, acc.at[pl.ds(s * rows_per_tile,
                                           rows_per_tile)])
        plsc.subcore_barrier()

        pltpu.sync_copy(vh_hbm.at[pl.ds(0, n_pad)], vloc)
        pltpu.sync_copy(vh_hbm.at[pl.ds(n_pad, n_pad)], vtmp)
        pltpu.sync_copy(src_hbm.at[s].at[pl.ds(c * nch2, nch2)], src_v)
        pltpu.sync_copy(dst_hbm.at[s].at[pl.ds(c * nch2, nch2)], dst_v)
        pltpu.sync_copy(adj_hbm.at[s].at[pl.ds(c * nch2, nch2)], adj_v)

        @pl.loop(0, n_pad // L)
        def _vsum(i):
            vloc[pl.ds(i * L, L)] = (vloc[pl.ds(i * L, L)]
                                     + vtmp[pl.ds(i * L, L)])

        @pl.loop(0, nch2)
        def _edges(ch):
            @pl.loop(0, CHUNK // L)
            def _msg(k):
                idx = src_v.at[ch][pl.ds(k * L, L)]
                vals = plsc.load_gather(vloc, [idx])
                msg_v.at[ch][pl.ds(k * L, L)] = (
                    vals * adj_v.at[ch][pl.ds(k * L, L)])
            pltpu.sync_copy(msg_v.at[ch], acc.at[dst_v.at[ch]], add=True)
        plsc.subcore_barrier()

        pltpu.sync_copy(acc.at[pl.ds(s * rows_per_tile, rows_per_tile)],
                        obuf)
        pltpu.sync_copy(obuf, out_hbm.at[c, pl.ds(s * rows_per_tile,
                                                  rows_per_tile)])

    return spmm2(src3, dst3, adj3, vh)


def kernel(edge_index, adj_values, feature, W1, b1, W2, b2, Wlin, blin):
    n = feature.shape[0]
    e = edge_index.shape[1]
    d1 = W1.shape[1]

    rows_per_tile = -(-n // (NS * CHUNK)) * CHUNK
    n_pad = NS * rows_per_tile
    e_pad = -(-e // (NW * 4 * CHUNK)) * NW * 4 * CHUNK
    nchunk16 = e_pad // (NS * CHUNK)
    nchunk32 = e_pad // (NW * CHUNK)
    d1h = d1 // 2

    src = edge_index[0].astype(jnp.int32)
    dst = edge_index[1].astype(jnp.int32)
    adv = adj_values.astype(jnp.float32)

    sup2, w2l, src_p, dst_p, adj_p = _tc_front(
        feature, W1, W2, Wlin, src, dst, adv, n_pad, e_pad)
    src16 = src_p.reshape(NS, nchunk16, CHUNK)
    dst16 = dst_p.reshape(NS, nchunk16, CHUNK)
    adj16 = adj_p.reshape(NS, nchunk16, CHUNK)

    vh = _sc_spmm_wide(src16, dst16, adj16, sup2, b1, w2l,
                       n_pad, d1h, nchunk16)
    q = _sc_spmm_scalar(src16, dst16, adj16, vh, n_pad, nchunk16)
    return _tc_final(q, b2, Wlin, blin, n)
